# baseline (speedup 1.0000x reference)
import functools

import jax
import jax.numpy as jnp
from jax import lax
from jax.experimental import pallas as pl
from jax.experimental.pallas import tpu as pltpu
from jax.experimental.pallas import tpu_sc as plsc

VOCAB = 1000000
DIM = 64
BATCH = 16384
NEG = 20
K1 = NEG + 1
LANES = 16
NC = 2
NS = 16
NW = NC * NS
B_PER_W = BATCH // NW
CB = 16
NCHUNK = B_PER_W // CB
KROWS = CB * K1
KSPLIT = 3
KG = KROWS // KSPLIT
CBK = K1 * LANES
NGROUPS = BATCH // CB
ROWW = 2 * DIM
VROWS = VOCAB // 2
PW = 256
PROWS = PW // 2
NPAN = VOCAB // PW
PPT = NPAN // NW + 1
TAILW = VOCAB - NPAN * PW


def _relayout_body(vt_in_hbm, vt_out_hbm, in2_hbm, out2_hbm,
                   in_bufs, out_bufs, tin_buf, tout_buf,
                   sem_i0, sem_i1, sem_o0, sem_o1, sem_t):
    wid = lax.axis_index("s") * NC + lax.axis_index("c")
    lane_iota = lax.iota(jnp.int32, LANES)
    half_lane = lane_iota >> 1
    parity64 = (lane_iota & 1) * DIM
    sem_i = (sem_i0, sem_i1)
    sem_o = (sem_o0, sem_o1)

    def transpose_panel(b):
        in_buf = in_bufs.at[b]
        out_buf = out_bufs.at[b]
        rowcs = [half_lane + m0 * 8 for m0 in range(PW // LANES)]
        DU = 8

        def d_body(d8, carry):
            d0 = d8 * DU
            for dd in range(DU):
                colv = parity64 + (d0 + dd)
                for m0 in range(PW // LANES):
                    val = in_buf[d0 + dd, pl.ds(m0 * LANES, LANES)]
                    plsc.store_scatter(out_buf, [rowcs[m0], colv], val)
            return carry

        lax.fori_loop(0, DIM // DU, d_body, 0)

    def start_in(src_hbm, g, b):
        for td in range(DIM // 8):
            pltpu.async_copy(
                src_hbm.at[pl.ds(td * 8, 8), pl.ds(g * PW, PW)],
                in_bufs.at[b].at[pl.ds(td * 8, 8)], sem_i[b])

    def wait_in(src_hbm, b):
        for td in range(DIM // 8):
            pltpu.make_async_copy(
                src_hbm.at[pl.ds(td * 8, 8), pl.ds(0, PW)],
                in_bufs.at[b].at[pl.ds(td * 8, 8)], sem_i[b]).wait()

    for src_hbm, dst_hbm in ((vt_in_hbm, in2_hbm), (vt_out_hbm, out2_hbm)):
        start_in(src_hbm, wid, 0)

        def pair_body(jj, carry, src_hbm=src_hbm, dst_hbm=dst_hbm):
            for b in range(2):
                j = jj * 2 + b
                g = j * NW + wid

                @pl.when(g < NPAN)
                def _(j=j, g=g, b=b):
                    gn = (j + 1) * NW + wid

                    @pl.when(gn < NPAN)
                    def _():
                        start_in(src_hbm, gn, 1 - b)

                    wait_in(src_hbm, b)

                    @pl.when(j >= 2)
                    def _():
                        pltpu.make_async_copy(
                            out_bufs.at[b],
                            dst_hbm.at[pl.ds(0, PROWS)], sem_o[b]).wait()

                    transpose_panel(b)
                    pltpu.async_copy(
                        out_bufs.at[b],
                        dst_hbm.at[pl.ds(g * PROWS, PROWS)], sem_o[b])

            return carry

        lax.fori_loop(0, (PPT + 1) // 2, pair_body, 0)
        for b in range(2):
            pltpu.make_async_copy(
                out_bufs.at[b], dst_hbm.at[pl.ds(0, PROWS)], sem_o[b]).wait()

    for t, (src_hbm, dst_hbm) in enumerate(
            ((vt_in_hbm, in2_hbm), (vt_out_hbm, out2_hbm))):
        @pl.when(wid == NW - 1 - t)
        def _(src_hbm=src_hbm, dst_hbm=dst_hbm):
            pltpu.async_copy(
                src_hbm.at[pl.ds(0, DIM), pl.ds(NPAN * PW, TAILW)],
                tin_buf, sem_t).wait()
            trowcs = [half_lane + m0 * 8 for m0 in range(TAILW // LANES)]

            def d_body(d, carry):
                colv = parity64 + d
                for m0 in range(TAILW // LANES):
                    val = tin_buf[d, pl.ds(m0 * LANES, LANES)]
                    plsc.store_scatter(tout_buf, [trowcs[m0], colv], val)
                return carry

            lax.fori_loop(0, DIM, d_body, 0)
            pltpu.async_copy(
                tout_buf, dst_hbm.at[pl.ds(NPAN * PROWS, TAILW // 2)],
                sem_t).wait()


_sc_relayout = functools.partial(
    pl.kernel,
    out_type=(jax.ShapeDtypeStruct((VROWS, ROWW), jnp.float32),
              jax.ShapeDtypeStruct((VROWS, ROWW), jnp.float32)),
    mesh=plsc.VectorSubcoreMesh(core_axis_name="c", subcore_axis_name="s"),
    compiler_params=pltpu.CompilerParams(
        needs_layout_passes=False, use_tc_tiling_on_sc=True),
    scratch_types=[
        pltpu.VMEM((2, DIM, PW), jnp.float32),
        pltpu.VMEM((2, PROWS, ROWW), jnp.float32),
        pltpu.VMEM((DIM, TAILW), jnp.float32),
        pltpu.VMEM((TAILW // 2, ROWW), jnp.float32),
        pltpu.SemaphoreType.DMA,
        pltpu.SemaphoreType.DMA,
        pltpu.SemaphoreType.DMA,
        pltpu.SemaphoreType.DMA,
        pltpu.SemaphoreType.DMA,
    ],
)(_relayout_body)


def _sc_body(cv_hbm, kv_hbm, ch_hbm, kh_hbm, in_emb_hbm, out_emb_hbm,
             scores_hbm, cidx_v, kidx_v, chh_v, khh_v, crow_v, krow_v,
             scores_v, sem):
    wid = lax.axis_index("s") * NC + lax.axis_index("c")
    lane_iota = lax.iota(jnp.int32, LANES)

    def chunk_body(c, carry):
        base = wid * B_PER_W + c * CB
        pltpu.sync_copy(cv_hbm.at[pl.ds(base, CB)], cidx_v)
        pltpu.sync_copy(kv_hbm.at[pl.ds(base * K1, KROWS)], kidx_v)
        pltpu.sync_copy(ch_hbm.at[pl.ds(base, CB)], chh_v.at[pl.ds(0, CB)])
        pltpu.sync_copy(kh_hbm.at[pl.ds(base * K1, KROWS)],
                        khh_v.at[pl.ds(0, KROWS)])
        handles = [pltpu.async_copy(in_emb_hbm.at[cidx_v], crow_v, sem)]
        for j in range(KSPLIT):
            handles.append(pltpu.async_copy(
                out_emb_hbm.at[kidx_v.at[pl.ds(j * KG, KG)]],
                krow_v.at[pl.ds(j * KG, KG)], sem))
        for h in handles:
            h.wait()

        def item_body(i, vecs):
            hc = chh_v[pl.ds(i, LANES)][0] * DIM
            cs = [crow_v[i, pl.ds(hc + q * LANES, LANES)]
                  for q in range(DIM // LANES)]
            out = []
            for k in range(K1):
                r = i * K1 + k
                hw = khh_v[pl.ds(r, LANES)][0] * DIM
                acc = cs[0] * krow_v[r, pl.ds(hw, LANES)]
                for q in range(1, DIM // LANES):
                    acc = acc + cs[q] * krow_v[r, pl.ds(hw + q * LANES, LANES)]
                s = jnp.sum(acc)
                out.append(jnp.where(lane_iota == i, s, vecs[k]))
            return tuple(out)

        vecs = lax.fori_loop(
            0, CB, item_body,
            tuple(jnp.zeros((LANES,), jnp.float32) for _ in range(K1)))
        for k in range(K1):
            scores_v[pl.ds(k * LANES, LANES)] = vecs[k]
        pltpu.sync_copy(scores_v, scores_hbm.at[wid * NCHUNK + c])
        return carry

    lax.fori_loop(0, NCHUNK, chunk_body, 0)


_sc_scores = functools.partial(
    pl.kernel,
    out_type=jax.ShapeDtypeStruct((NGROUPS, CBK), jnp.float32),
    mesh=plsc.VectorSubcoreMesh(core_axis_name="c", subcore_axis_name="s"),
    compiler_params=pltpu.CompilerParams(
        needs_layout_passes=False, use_tc_tiling_on_sc=True),
    scratch_types=[
        pltpu.VMEM((CB,), jnp.int32),
        pltpu.VMEM((KROWS,), jnp.int32),
        pltpu.VMEM((CB + LANES,), jnp.int32),
        pltpu.VMEM((KROWS + LANES,), jnp.int32),
        pltpu.VMEM((CB, ROWW), jnp.float32),
        pltpu.VMEM((KROWS, ROWW), jnp.float32),
        pltpu.VMEM((CBK,), jnp.float32),
        pltpu.SemaphoreType.DMA,
    ],
)(_sc_body)


def _tc_loss_body(scores_ref, out_ref):
    x = scores_ref[...]
    r = lax.broadcasted_iota(jnp.int32, x.shape, 0)
    c = lax.broadcasted_iota(jnp.int32, x.shape, 1)
    k = (r * (x.shape[1] // LANES) + c // LANES) % K1
    z = jnp.where(k == 0, x, -x)
    loss = -jnp.log(jax.nn.sigmoid(z) + 1e-10)
    out_ref[0, 0] = jnp.sum(loss) * (1.0 / BATCH)


def kernel(center_words, context_words, negative_samples, in_emb, out_emb):
    center = center_words.astype(jnp.int32)
    combo = jnp.concatenate(
        [context_words[:, None], negative_samples], axis=1
    ).reshape(-1).astype(jnp.int32)
    in2, out2 = _sc_relayout(in_emb.T, out_emb.T)
    scores = _sc_scores(center >> 1, combo >> 1, center & 1, combo & 1,
                        in2, out2)
    flat = scores.reshape(NGROUPS * CBK // 128, 128)
    loss = pl.pallas_call(
        _tc_loss_body,
        out_shape=jax.ShapeDtypeStruct((1, 1), jnp.float32),
        out_specs=pl.BlockSpec(memory_space=pltpu.SMEM),
    )(flat)
    return loss[0, 0]


# speedup vs baseline: 1.6400x; 1.6400x over previous
import functools

import jax
import jax.numpy as jnp
from jax import lax
from jax.experimental import pallas as pl
from jax.experimental.pallas import tpu as pltpu
from jax.experimental.pallas import tpu_sc as plsc

VOCAB = 1000000
DIM = 64
BATCH = 16384
NEG = 20
K1 = NEG + 1
LANES = 16
NC = 2
NS = 16
NW = NC * NS
B_PER_W = BATCH // NW
CB = 16
NCHUNK = B_PER_W // CB
KROWS = CB * K1
KSPLIT = 3
KG = KROWS // KSPLIT
CBK = K1 * LANES
NGROUPS = BATCH // CB
ROWW = 2 * DIM
VROWS = VOCAB // 2
TBW = 1024
TBH = TBW // 2
TGRID = (VOCAB + TBW - 1) // TBW
VROWS2 = TGRID * TBH


def _tc_transpose_body(x_ref, y_ref):
    x = x_ref[...]
    y_ref[...] = jnp.concatenate(
        [x[:, :TBH].T, x[:, TBH:].T], axis=1)


_tc_transpose = pl.pallas_call(
    _tc_transpose_body,
    grid=(TGRID,),
    in_specs=[pl.BlockSpec((DIM, TBW), lambda j: (0, j))],
    out_specs=pl.BlockSpec((TBH, ROWW), lambda j: (j, 0)),
    out_shape=jax.ShapeDtypeStruct((VROWS2, ROWW), jnp.float32),
)


def _sc_body(cv_hbm, kv_hbm, ch_hbm, kh_hbm, in_emb_hbm, out_emb_hbm,
             scores_hbm, cidx_v, kidx_v, chh_v, khh_v, crow_v, krow_v,
             scores_v, sem):
    wid = lax.axis_index("s") * NC + lax.axis_index("c")
    lane_iota = lax.iota(jnp.int32, LANES)

    def chunk_body(c, carry):
        base = wid * B_PER_W + c * CB
        pltpu.sync_copy(cv_hbm.at[pl.ds(base, CB)], cidx_v)
        pltpu.sync_copy(kv_hbm.at[pl.ds(base * K1, KROWS)], kidx_v)
        pltpu.sync_copy(ch_hbm.at[pl.ds(base, CB)], chh_v.at[pl.ds(0, CB)])
        pltpu.sync_copy(kh_hbm.at[pl.ds(base * K1, KROWS)],
                        khh_v.at[pl.ds(0, KROWS)])
        handles = [pltpu.async_copy(in_emb_hbm.at[cidx_v], crow_v, sem)]
        for j in range(KSPLIT):
            handles.append(pltpu.async_copy(
                out_emb_hbm.at[kidx_v.at[pl.ds(j * KG, KG)]],
                krow_v.at[pl.ds(j * KG, KG)], sem))
        for h in handles:
            h.wait()

        def item_body(i, vecs):
            hc = chh_v[pl.ds(i, LANES)][0] * DIM
            cs = [crow_v[i, pl.ds(hc + q * LANES, LANES)]
                  for q in range(DIM // LANES)]
            out = []
            for k in range(K1):
                r = i * K1 + k
                hw = khh_v[pl.ds(r, LANES)][0] * DIM
                acc = cs[0] * krow_v[r, pl.ds(hw, LANES)]
                for q in range(1, DIM // LANES):
                    acc = acc + cs[q] * krow_v[r, pl.ds(hw + q * LANES, LANES)]
                s = jnp.sum(acc)
                out.append(jnp.where(lane_iota == i, s, vecs[k]))
            return tuple(out)

        vecs = lax.fori_loop(
            0, CB, item_body,
            tuple(jnp.zeros((LANES,), jnp.float32) for _ in range(K1)))
        for k in range(K1):
            scores_v[pl.ds(k * LANES, LANES)] = vecs[k]
        pltpu.sync_copy(scores_v, scores_hbm.at[wid * NCHUNK + c])
        return carry

    lax.fori_loop(0, NCHUNK, chunk_body, 0)


_sc_scores = functools.partial(
    pl.kernel,
    out_type=jax.ShapeDtypeStruct((NGROUPS, CBK), jnp.float32),
    mesh=plsc.VectorSubcoreMesh(core_axis_name="c", subcore_axis_name="s"),
    compiler_params=pltpu.CompilerParams(
        needs_layout_passes=False, use_tc_tiling_on_sc=True),
    scratch_types=[
        pltpu.VMEM((CB,), jnp.int32),
        pltpu.VMEM((KROWS,), jnp.int32),
        pltpu.VMEM((CB + LANES,), jnp.int32),
        pltpu.VMEM((KROWS + LANES,), jnp.int32),
        pltpu.VMEM((CB, ROWW), jnp.float32),
        pltpu.VMEM((KROWS, ROWW), jnp.float32),
        pltpu.VMEM((CBK,), jnp.float32),
        pltpu.SemaphoreType.DMA,
    ],
)(_sc_body)


def _tc_loss_body(scores_ref, out_ref):
    x = scores_ref[...]
    r = lax.broadcasted_iota(jnp.int32, x.shape, 0)
    c = lax.broadcasted_iota(jnp.int32, x.shape, 1)
    k = (r * (x.shape[1] // LANES) + c // LANES) % K1
    z = jnp.where(k == 0, x, -x)
    loss = -jnp.log(jax.nn.sigmoid(z) + 1e-10)
    out_ref[0, 0] = jnp.sum(loss) * (1.0 / BATCH)


def kernel(center_words, context_words, negative_samples, in_emb, out_emb):
    center = center_words.astype(jnp.int32)
    combo = jnp.concatenate(
        [context_words[:, None], negative_samples], axis=1
    ).reshape(-1).astype(jnp.int32)
    in2 = _tc_transpose(in_emb.T)
    out2 = _tc_transpose(out_emb.T)

    def row_of(w):
        return ((w >> 10) << 9) + (w & (TBH - 1))

    def half_of(w):
        return (w >> 9) & 1

    scores = _sc_scores(row_of(center), row_of(combo),
                        half_of(center), half_of(combo), in2, out2)
    flat = scores.reshape(NGROUPS * CBK // 128, 128)
    loss = pl.pallas_call(
        _tc_loss_body,
        out_shape=jax.ShapeDtypeStruct((1, 1), jnp.float32),
        out_specs=pl.BlockSpec(memory_space=pltpu.SMEM),
    )(flat)
    return loss[0, 0]


# speedup vs baseline: 3.0084x; 1.8344x over previous
import functools

import jax
import jax.numpy as jnp
from jax import lax
from jax.experimental import pallas as pl
from jax.experimental.pallas import tpu as pltpu
from jax.experimental.pallas import tpu_sc as plsc

VOCAB = 1000000
DIM = 64
BATCH = 16384
NEG = 20
K1 = NEG + 1
LANES = 16
NC = 2
NS = 16
NW = NC * NS
B_PER_W = BATCH // NW
CB = 16
NCHUNK = B_PER_W // CB
KROWS = CB * K1
KSPLIT = 3
KG = KROWS // KSPLIT
CBK = K1 * LANES
NGROUPS = BATCH // CB
ROWW = 2 * DIM
VROWS = VOCAB // 2
TBW = 4096
TBH = TBW // 2
TGRID = (VOCAB + TBW - 1) // TBW
VROWS2 = TGRID * TBH


def _tc_transpose_body(x_ref, y_ref):
    x = x_ref[...]
    y_ref[...] = jnp.concatenate(
        [x[:, :TBH].T, x[:, TBH:].T], axis=1)


_tc_transpose = pl.pallas_call(
    _tc_transpose_body,
    grid=(TGRID,),
    in_specs=[pl.BlockSpec((DIM, TBW), lambda j: (0, j))],
    out_specs=pl.BlockSpec((TBH, ROWW), lambda j: (j, 0)),
    out_shape=jax.ShapeDtypeStruct((VROWS2, ROWW), jnp.float32),
)


def _sc_body(cv_hbm, kv_hbm, ch_hbm, kh_hbm, in_emb_hbm, out_emb_hbm,
             scores_hbm, cidx_v, kidx_v, chh_v, khh_v, crow_v, krow_v,
             scores_v, sem):
    wid = lax.axis_index("s") * NC + lax.axis_index("c")
    lane_iota = lax.iota(jnp.int32, LANES)

    def chunk_body(c, carry):
        base = wid * B_PER_W + c * CB
        pltpu.sync_copy(cv_hbm.at[pl.ds(base, CB)], cidx_v)
        pltpu.sync_copy(kv_hbm.at[pl.ds(base * K1, KROWS)], kidx_v)
        pltpu.sync_copy(ch_hbm.at[pl.ds(base, CB)], chh_v.at[pl.ds(0, CB)])
        pltpu.sync_copy(kh_hbm.at[pl.ds(base * K1, KROWS)],
                        khh_v.at[pl.ds(0, KROWS)])
        handles = [pltpu.async_copy(in_emb_hbm.at[cidx_v], crow_v, sem)]
        for j in range(KSPLIT):
            handles.append(pltpu.async_copy(
                out_emb_hbm.at[kidx_v.at[pl.ds(j * KG, KG)]],
                krow_v.at[pl.ds(j * KG, KG)], sem))
        for h in handles:
            h.wait()

        def item_body(i, vecs):
            hc = chh_v[pl.ds(i, LANES)][0] * DIM
            cs = [crow_v[i, pl.ds(hc + q * LANES, LANES)]
                  for q in range(DIM // LANES)]
            out = []
            for k in range(K1):
                r = i * K1 + k
                hw = khh_v[pl.ds(r, LANES)][0] * DIM
                acc = cs[0] * krow_v[r, pl.ds(hw, LANES)]
                for q in range(1, DIM // LANES):
                    acc = acc + cs[q] * krow_v[r, pl.ds(hw + q * LANES, LANES)]
                s = jnp.sum(acc)
                out.append(jnp.where(lane_iota == i, s, vecs[k]))
            return tuple(out)

        vecs = lax.fori_loop(
            0, CB, item_body,
            tuple(jnp.zeros((LANES,), jnp.float32) for _ in range(K1)))
        for k in range(K1):
            scores_v[pl.ds(k * LANES, LANES)] = vecs[k]
        pltpu.sync_copy(scores_v, scores_hbm.at[wid * NCHUNK + c])
        return carry

    lax.fori_loop(0, NCHUNK, chunk_body, 0)


_sc_scores = functools.partial(
    pl.kernel,
    out_type=jax.ShapeDtypeStruct((NGROUPS, CBK), jnp.float32),
    mesh=plsc.VectorSubcoreMesh(core_axis_name="c", subcore_axis_name="s"),
    compiler_params=pltpu.CompilerParams(
        needs_layout_passes=False, use_tc_tiling_on_sc=True),
    scratch_types=[
        pltpu.VMEM((CB,), jnp.int32),
        pltpu.VMEM((KROWS,), jnp.int32),
        pltpu.VMEM((CB + LANES,), jnp.int32),
        pltpu.VMEM((KROWS + LANES,), jnp.int32),
        pltpu.VMEM((CB, ROWW), jnp.float32),
        pltpu.VMEM((KROWS, ROWW), jnp.float32),
        pltpu.VMEM((CBK,), jnp.float32),
        pltpu.SemaphoreType.DMA,
    ],
)(_sc_body)


def _tc_loss_body(scores_ref, out_ref):
    x = scores_ref[...]
    r = lax.broadcasted_iota(jnp.int32, x.shape, 0)
    c = lax.broadcasted_iota(jnp.int32, x.shape, 1)
    k = (r * (x.shape[1] // LANES) + c // LANES) % K1
    z = jnp.where(k == 0, x, -x)
    loss = -jnp.log(jax.nn.sigmoid(z) + 1e-10)
    out_ref[0, 0] = jnp.sum(loss) * (1.0 / BATCH)


def kernel(center_words, context_words, negative_samples, in_emb, out_emb):
    center = center_words.astype(jnp.int32)
    combo = jnp.concatenate(
        [context_words[:, None], negative_samples], axis=1
    ).reshape(-1).astype(jnp.int32)
    in2 = _tc_transpose(in_emb.T)
    out2 = _tc_transpose(out_emb.T)

    def row_of(w):
        return (w // TBW) * TBH + (w % TBH)

    def half_of(w):
        return (w // TBH) % 2

    scores = _sc_scores(row_of(center), row_of(combo),
                        half_of(center), half_of(combo), in2, out2)
    flat = scores.reshape(NGROUPS * CBK // 128, 128)
    loss = pl.pallas_call(
        _tc_loss_body,
        out_shape=jax.ShapeDtypeStruct((1, 1), jnp.float32),
        out_specs=pl.BlockSpec(memory_space=pltpu.SMEM),
    )(flat)
    return loss[0, 0]
